# Initial kernel scaffold; baseline (speedup 1.0000x reference)
import functools

import jax
import jax.numpy as jnp
from jax import lax
from jax.experimental import pallas as pl
from jax.experimental.pallas import tpu as pltpu
from jax.experimental.pallas import tpu_sc as plsc

_NUM_BLUEPRINTS = 1_000_000
_NULL_IDX = _NUM_BLUEPRINTS
_D = 32
_L = 16
_SZ = 128
_ROWS_PER_CHUNK = 8
_CHUNK = _SZ * _ROWS_PER_CHUNK
_NC = 2
_NS = 16
_NW = _NC * _NS


def _make_kernel(n_rows):
    rows_per_w = n_rows // _NW
    chunks_per_w = rows_per_w // _ROWS_PER_CHUNK
    pairs = chunks_per_w // 2
    n_total = n_rows * _SZ

    mesh = plsc.VectorSubcoreMesh(
        core_axis_name="c", subcore_axis_name="s",
        num_cores=_NC, num_subcores=_NS)

    @functools.partial(
        pl.kernel,
        out_type=jax.ShapeDtypeStruct((n_total, _D), jnp.float32),
        mesh=mesh,
        scratch_types=[
            pltpu.VMEM((2, _ROWS_PER_CHUNK, _SZ), jnp.int32),
            pltpu.VMEM((2, _CHUNK, _D), jnp.float32),
            pltpu.SemaphoreType.DMA,
            pltpu.SemaphoreType.DMA,
            pltpu.SemaphoreType.DMA,
            pltpu.SemaphoreType.DMA,
        ],
    )
    def k(idx_hbm, table_hbm, out_hbm, idx_v, rows_v, g0, g1, s0, s1):
        wid = lax.axis_index("s") * _NC + lax.axis_index("c")
        row_base = wid * rows_per_w
        gsems = (g0, g1)
        ssems = (s0, s1)

        def remap(slot):
            def body(j, carry):
                for l in range(_SZ // _L):
                    v = idx_v[slot, j, pl.ds(l * _L, _L)]
                    v = jnp.where(v < 0, jnp.int32(_NULL_IDX), v)
                    idx_v[slot, j, pl.ds(l * _L, _L)] = v
                return carry
            lax.fori_loop(0, _ROWS_PER_CHUNK, body, 0)

        def issue(g, slot):
            row = row_base + g * _ROWS_PER_CHUNK
            pltpu.sync_copy(idx_hbm.at[pl.ds(row, _ROWS_PER_CHUNK)],
                            idx_v.at[slot])
            remap(slot)
            return [
                pltpu.async_copy(
                    table_hbm.at[idx_v.at[slot, j]],
                    rows_v.at[slot, pl.ds(j * _SZ, _SZ)],
                    gsems[slot])
                for j in range(_ROWS_PER_CHUNK)
            ]

        def store(g, slot):
            out_row = (row_base + g * _ROWS_PER_CHUNK) * _SZ
            return pltpu.async_copy(rows_v.at[slot],
                                    out_hbm.at[pl.ds(out_row, _CHUNK)],
                                    ssems[slot])

        def pair_body(i, carry):
            ga = 2 * i
            gb = 2 * i + 1
            cps0 = issue(ga, 0)
            cps1 = issue(gb, 1)
            for cp in cps0:
                cp.wait()
            st0 = store(ga, 0)
            for cp in cps1:
                cp.wait()
            st1 = store(gb, 1)
            st0.wait()
            st1.wait()
            return carry

        lax.fori_loop(0, pairs, pair_body, 0)

    return k


def kernel(blueprint_indices, embedding_weight):
    b, s = blueprint_indices.shape
    n = b * s
    idx2d = blueprint_indices.astype(jnp.int32).reshape(n // _SZ, _SZ)
    out = _make_kernel(n // _SZ)(idx2d, embedding_weight)
    return out.reshape(b, s, _D)


# speedup vs baseline: 1.1095x; 1.1095x over previous
import functools

import jax
import jax.numpy as jnp
from jax import lax
from jax.experimental import pallas as pl
from jax.experimental.pallas import tpu as pltpu
from jax.experimental.pallas import tpu_sc as plsc

_NUM_BLUEPRINTS = 1_000_000
_NULL_IDX = _NUM_BLUEPRINTS
_D = 32
_L = 16
_SZ = 128
_ROWS_PER_CHUNK = 8
_CHUNK = _SZ * _ROWS_PER_CHUNK
_NC = 2
_NS = 16
_NW = _NC * _NS


def _make_kernel(n_rows):
    rows_per_w = n_rows // _NW
    chunks_per_w = rows_per_w // _ROWS_PER_CHUNK
    pairs = chunks_per_w // 2
    n_total = n_rows * _SZ

    mesh = plsc.VectorSubcoreMesh(
        core_axis_name="c", subcore_axis_name="s",
        num_cores=_NC, num_subcores=_NS)

    @functools.partial(
        pl.kernel,
        out_type=jax.ShapeDtypeStruct((n_total, _D), jnp.float32),
        mesh=mesh,
        compiler_params=pltpu.CompilerParams(use_tc_tiling_on_sc=False),
        scratch_types=[
            pltpu.VMEM((2, _ROWS_PER_CHUNK, _SZ), jnp.int32),
            pltpu.VMEM((2, _CHUNK, _D), jnp.float32),
            pltpu.SemaphoreType.DMA,
            pltpu.SemaphoreType.DMA,
            pltpu.SemaphoreType.DMA,
            pltpu.SemaphoreType.DMA,
        ],
    )
    def k(idx_hbm, table_hbm, out_hbm, idx_v, rows_v, g0, g1, s0, s1):
        wid = lax.axis_index("s") * _NC + lax.axis_index("c")
        row_base = wid * rows_per_w
        gsems = (g0, g1)
        ssems = (s0, s1)

        def remap(slot):
            def body(j, carry):
                for l in range(_SZ // _L):
                    v = idx_v[slot, j, pl.ds(l * _L, _L)]
                    v = jnp.where(v < 0, jnp.int32(_NULL_IDX), v)
                    idx_v[slot, j, pl.ds(l * _L, _L)] = v
                return carry
            lax.fori_loop(0, _ROWS_PER_CHUNK, body, 0)

        def issue(g, slot):
            row = row_base + g * _ROWS_PER_CHUNK
            pltpu.sync_copy(idx_hbm.at[pl.ds(row, _ROWS_PER_CHUNK)],
                            idx_v.at[slot])
            remap(slot)
            return [
                pltpu.async_copy(
                    table_hbm.at[idx_v.at[slot, j]],
                    rows_v.at[slot, pl.ds(j * _SZ, _SZ)],
                    gsems[slot])
                for j in range(_ROWS_PER_CHUNK)
            ]

        def store(g, slot):
            out_row = (row_base + g * _ROWS_PER_CHUNK) * _SZ
            return pltpu.async_copy(rows_v.at[slot],
                                    out_hbm.at[pl.ds(out_row, _CHUNK)],
                                    ssems[slot])

        def pair_body(i, carry):
            ga = 2 * i
            gb = 2 * i + 1
            cps0 = issue(ga, 0)
            cps1 = issue(gb, 1)
            for cp in cps0:
                cp.wait()
            st0 = store(ga, 0)
            for cp in cps1:
                cp.wait()
            st1 = store(gb, 1)
            st0.wait()
            st1.wait()
            return carry

        lax.fori_loop(0, pairs, pair_body, 0)

    return k


def kernel(blueprint_indices, embedding_weight):
    b, s = blueprint_indices.shape
    n = b * s
    idx2d = blueprint_indices.astype(jnp.int32).reshape(n // _SZ, _SZ)
    out = _make_kernel(n // _SZ)(idx2d, embedding_weight)
    return out.reshape(b, s, _D)


# speedup vs baseline: 4.3755x; 3.9439x over previous
import functools

import jax
import jax.numpy as jnp
from jax import lax
from jax.experimental import pallas as pl
from jax.experimental.pallas import tpu as pltpu
from jax.experimental.pallas import tpu_sc as plsc

_NUM_BLUEPRINTS = 1_000_000
_NULL_IDX = _NUM_BLUEPRINTS
_D = 32
_L = 16
_SLABS_PER_CHUNK = 8
_NC = 2
_NS = 16
_NW = _NC * _NS


def _make_kernel(n_slabs, s):
    slabs_per_w = n_slabs // _NW
    chunks_per_w = slabs_per_w // _SLABS_PER_CHUNK
    pairs = chunks_per_w // 2

    mesh = plsc.VectorSubcoreMesh(
        core_axis_name="c", subcore_axis_name="s",
        num_cores=_NC, num_subcores=_NS)

    @functools.partial(
        pl.kernel,
        out_type=jax.ShapeDtypeStruct((n_slabs, s, _D), jnp.float32),
        mesh=mesh,
        compiler_params=pltpu.CompilerParams(use_tc_tiling_on_sc=False),
        scratch_types=[
            pltpu.VMEM((2, _SLABS_PER_CHUNK, s), jnp.int32),
            pltpu.VMEM((2, _SLABS_PER_CHUNK, s, _D), jnp.float32),
            pltpu.SemaphoreType.DMA,
            pltpu.SemaphoreType.DMA,
            pltpu.SemaphoreType.DMA,
            pltpu.SemaphoreType.DMA,
        ],
    )
    def k(idx_hbm, table_hbm, out_hbm, idx_v, rows_v, g0, g1, s0, s1):
        wid = lax.axis_index("s") * _NC + lax.axis_index("c")
        slab_base = wid * slabs_per_w
        gsems = (g0, g1)
        ssems = (s0, s1)

        offs = list(range(0, s - _L + 1, _L))
        if s % _L != 0:
            offs.append(s - _L)

        def remap(slot):
            def body(j, carry):
                for o in offs:
                    v = idx_v[slot, j, pl.ds(o, _L)]
                    v = jnp.where(v < 0, jnp.int32(_NULL_IDX), v)
                    idx_v[slot, j, pl.ds(o, _L)] = v
                return carry
            lax.fori_loop(0, _SLABS_PER_CHUNK, body, 0)

        def issue(g, slot):
            slab = slab_base + g * _SLABS_PER_CHUNK
            pltpu.sync_copy(idx_hbm.at[pl.ds(slab, _SLABS_PER_CHUNK)],
                            idx_v.at[slot])
            remap(slot)
            return [
                pltpu.async_copy(
                    table_hbm.at[idx_v.at[slot, j]],
                    rows_v.at[slot, j],
                    gsems[slot])
                for j in range(_SLABS_PER_CHUNK)
            ]

        def store(g, slot):
            slab = slab_base + g * _SLABS_PER_CHUNK
            return pltpu.async_copy(rows_v.at[slot],
                                    out_hbm.at[pl.ds(slab, _SLABS_PER_CHUNK)],
                                    ssems[slot])

        def pair_body(i, carry):
            ga = 2 * i
            gb = 2 * i + 1
            cps0 = issue(ga, 0)
            cps1 = issue(gb, 1)
            for cp in cps0:
                cp.wait()
            st0 = store(ga, 0)
            for cp in cps1:
                cp.wait()
            st1 = store(gb, 1)
            st0.wait()
            st1.wait()
            return carry

        lax.fori_loop(0, pairs, pair_body, 0)

    return k


def kernel(blueprint_indices, embedding_weight):
    b, s = blueprint_indices.shape
    idx = blueprint_indices.astype(jnp.int32)
    return _make_kernel(b, s)(idx, embedding_weight)
